# baseline (speedup 1.0000x reference)
import functools

import jax
import jax.numpy as jnp
from jax import lax
from jax.experimental import pallas as pl
from jax.experimental.pallas import tpu as pltpu
from jax.experimental.pallas import tpu_sc as plsc

_N_FIELDS = 26
_VOCAB = 100000
_EMB_DIM = 32
_BATCH = 16384
_N_NUM = 13
_BN_EPS = 1e-5

_NC = 2
_NS = 16
_NW = _NC * _NS

_OUT_D = _N_FIELDS * _EMB_DIM + _N_NUM
_EMB_D = _N_FIELDS * _EMB_DIM

_ROWS = _BATCH * _N_FIELDS
_RPW = _ROWS // _NW
_CHUNK = 128
_CPW = _RPW // _CHUNK
_GROUP_CHUNKS = 4
_GROUP_ROWS = _CHUNK * _GROUP_CHUNKS
_N_GROUPS = _RPW // _GROUP_ROWS


def _sc_gather(tables_flat, idx):
    mesh = plsc.VectorSubcoreMesh(
        core_axis_name="c", subcore_axis_name="s",
        num_cores=_NC, num_subcores=_NS)

    @functools.partial(
        pl.kernel,
        out_type=jax.ShapeDtypeStruct((_ROWS, _EMB_DIM), jnp.float32),
        mesh=mesh,
        scratch_types=[
            pltpu.VMEM((_CPW, _CHUNK), jnp.int32),
            pltpu.VMEM((_GROUP_ROWS, _EMB_DIM), jnp.float32),
            pltpu.VMEM((_GROUP_ROWS, _EMB_DIM), jnp.float32),
            pltpu.SemaphoreType.DMA,
            pltpu.SemaphoreType.DMA,
        ],
        compiler_params=pltpu.CompilerParams(use_tc_tiling_on_sc=False),
    )
    def k(tbl_hbm, idx_hbm, out_hbm, idx_v, buf0, buf1, sem0, sem1):
        wid = lax.axis_index("s") * _NC + lax.axis_index("c")
        pltpu.sync_copy(idx_hbm.at[wid], idx_v)
        base_row = wid * _RPW

        bufs = (buf0, buf1)
        sems = (sem0, sem1)

        def fire(g, p):
            for j in range(_GROUP_CHUNKS):
                pltpu.async_copy(
                    tbl_hbm.at[idx_v.at[g * _GROUP_CHUNKS + j]],
                    bufs[p].at[pl.ds(j * _CHUNK, _CHUNK)],
                    sems[p])

        def drain(g, p):
            for j in range(_GROUP_CHUNKS):
                pltpu.make_async_copy(
                    tbl_hbm.at[idx_v.at[g * _GROUP_CHUNKS + j]],
                    bufs[p].at[pl.ds(j * _CHUNK, _CHUNK)],
                    sems[p]).wait()

        def wout(g, p):
            pltpu.sync_copy(
                bufs[p],
                out_hbm.at[pl.ds(base_row + g * _GROUP_ROWS, _GROUP_ROWS)])

        fire(0, 0)

        def group_pair(h, carry):
            g0 = 2 * h
            fire(g0 + 1, 1)
            drain(g0, 0)
            wout(g0, 0)

            @pl.when(h < _N_GROUPS // 2 - 1)
            def _():
                fire(g0 + 2, 0)

            drain(g0 + 1, 1)
            wout(g0 + 1, 1)
            return carry

        lax.fori_loop(0, _N_GROUPS // 2, group_pair, 0)

    return k(tables_flat, idx)


def _stats_body(x_ref, g_ref, b_ref, scale_ref, shift_ref, s_acc, q_acc):
    i = pl.program_id(0)

    @pl.when(i == 0)
    def _():
        s_acc[...] = jnp.zeros_like(s_acc)
        q_acc[...] = jnp.zeros_like(q_acc)

    x = x_ref[...]
    s_acc[...] += jnp.sum(x, axis=0)
    q_acc[...] += jnp.sum(x * x, axis=0)

    @pl.when(i == pl.num_programs(0) - 1)
    def _():
        n = float(_BATCH)
        mean = s_acc[...] / n
        var = q_acc[...] / n - mean * mean
        scale = g_ref[...] * lax.rsqrt(var + _BN_EPS)
        scale_ref[...] = scale
        shift_ref[...] = b_ref[...] - mean * scale


def _bn_stats(x_numerical, gamma, beta):
    grid = 8
    rows = _BATCH // grid
    return pl.pallas_call(
        _stats_body,
        grid=(grid,),
        in_specs=[
            pl.BlockSpec((rows, _N_NUM), lambda i: (i, 0)),
            pl.BlockSpec((_N_NUM,), lambda i: (0,)),
            pl.BlockSpec((_N_NUM,), lambda i: (0,)),
        ],
        out_specs=[
            pl.BlockSpec((_N_NUM,), lambda i: (0,)),
            pl.BlockSpec((_N_NUM,), lambda i: (0,)),
        ],
        out_shape=[
            jax.ShapeDtypeStruct((_N_NUM,), jnp.float32),
            jax.ShapeDtypeStruct((_N_NUM,), jnp.float32),
        ],
        scratch_shapes=[
            pltpu.VMEM((_N_NUM,), jnp.float32),
            pltpu.VMEM((_N_NUM,), jnp.float32),
        ],
    )(x_numerical, gamma, beta)


def _asm_body(emb_ref, x_ref, s_ref, t_ref, o_ref):
    o_ref[:, pl.ds(0, _EMB_D)] = emb_ref[...]
    o_ref[:, pl.ds(_EMB_D, _N_NUM)] = x_ref[...] * s_ref[...] + t_ref[...]


def _assemble(emb, x_numerical, scale, shift):
    grid = 32
    rows = _BATCH // grid
    return pl.pallas_call(
        _asm_body,
        grid=(grid,),
        in_specs=[
            pl.BlockSpec((rows, _EMB_D), lambda i: (i, 0)),
            pl.BlockSpec((rows, _N_NUM), lambda i: (i, 0)),
            pl.BlockSpec((_N_NUM,), lambda i: (0,)),
            pl.BlockSpec((_N_NUM,), lambda i: (0,)),
        ],
        out_specs=pl.BlockSpec((rows, _OUT_D), lambda i: (i, 0)),
        out_shape=jax.ShapeDtypeStruct((_BATCH, _OUT_D), jnp.float32),
    )(emb, x_numerical, scale, shift)


def kernel(x_numerical, x_cat, tables, gamma, beta):
    idx = (x_cat.astype(jnp.int32)
           + jnp.arange(_N_FIELDS, dtype=jnp.int32) * _VOCAB)
    idx = idx.reshape(_NW, _CPW, _CHUNK)
    tables_flat = tables.reshape(_N_FIELDS * _VOCAB, _EMB_DIM)

    emb = _sc_gather(tables_flat, idx).reshape(_BATCH, _EMB_D)
    scale, shift = _bn_stats(x_numerical, gamma, beta)
    return _assemble(emb, x_numerical, scale, shift)


# speedup vs baseline: 1.0344x; 1.0344x over previous
import functools

import jax
import jax.numpy as jnp
from jax import lax
from jax.experimental import pallas as pl
from jax.experimental.pallas import tpu as pltpu
from jax.experimental.pallas import tpu_sc as plsc

_N_FIELDS = 26
_VOCAB = 100000
_EMB_DIM = 32
_BATCH = 16384
_N_NUM = 13
_BN_EPS = 1e-5

_NC = 2
_NS = 16
_NW = _NC * _NS

_OUT_D = _N_FIELDS * _EMB_DIM + _N_NUM
_EMB_D = _N_FIELDS * _EMB_DIM

_STRIP_B = 1024
_SPF = _BATCH // _STRIP_B
_N_STRIPS = _N_FIELDS * _SPF
_SPW = _N_STRIPS // _NW
_CHUNK = 128
_CPS = _STRIP_B // _CHUNK
_BPW = _BATCH // _NW


def _stats_body(x_ref, g_ref, b_ref, scale_ref, shift_ref, s_acc, q_acc):
    i = pl.program_id(0)

    @pl.when(i == 0)
    def _():
        s_acc[...] = jnp.zeros_like(s_acc)
        q_acc[...] = jnp.zeros_like(q_acc)

    x = x_ref[...]
    s_acc[...] += jnp.sum(x, axis=0)
    q_acc[...] += jnp.sum(x * x, axis=0)

    @pl.when(i == pl.num_programs(0) - 1)
    def _():
        n = float(_BATCH)
        mean = s_acc[...] / n
        var = q_acc[...] / n - mean * mean
        scale = g_ref[...] * lax.rsqrt(var + _BN_EPS)
        scale_ref[...] = scale
        shift_ref[...] = b_ref[...] - mean * scale


def _bn_stats(x_numerical, gamma, beta):
    grid = 8
    rows = _BATCH // grid
    return pl.pallas_call(
        _stats_body,
        grid=(grid,),
        in_specs=[
            pl.BlockSpec((rows, _N_NUM), lambda i: (i, 0)),
            pl.BlockSpec((_N_NUM,), lambda i: (0,)),
            pl.BlockSpec((_N_NUM,), lambda i: (0,)),
        ],
        out_specs=[
            pl.BlockSpec((_N_NUM,), lambda i: (0,)),
            pl.BlockSpec((_N_NUM,), lambda i: (0,)),
        ],
        out_shape=[
            jax.ShapeDtypeStruct((_N_NUM,), jnp.float32),
            jax.ShapeDtypeStruct((_N_NUM,), jnp.float32),
        ],
        scratch_shapes=[
            pltpu.VMEM((_N_NUM,), jnp.float32),
            pltpu.VMEM((_N_NUM,), jnp.float32),
        ],
    )(x_numerical, gamma, beta)


def _apply_body(x_ref, s_ref, t_ref, o_ref):
    o_ref[...] = x_ref[...] * s_ref[...] + t_ref[...]


def _bn_apply(x_numerical, scale, shift):
    grid = 8
    rows = _BATCH // grid
    return pl.pallas_call(
        _apply_body,
        grid=(grid,),
        in_specs=[
            pl.BlockSpec((rows, _N_NUM), lambda i: (i, 0)),
            pl.BlockSpec((_N_NUM,), lambda i: (0,)),
            pl.BlockSpec((_N_NUM,), lambda i: (0,)),
        ],
        out_specs=pl.BlockSpec((rows, _N_NUM), lambda i: (i, 0)),
        out_shape=jax.ShapeDtypeStruct((_BATCH, _N_NUM), jnp.float32),
    )(x_numerical, scale, shift)


def _sc_fused(tables_flat, idx, cont):
    mesh = plsc.VectorSubcoreMesh(
        core_axis_name="c", subcore_axis_name="s",
        num_cores=_NC, num_subcores=_NS)

    @functools.partial(
        pl.kernel,
        out_type=jax.ShapeDtypeStruct((_BATCH, _OUT_D), jnp.float32),
        mesh=mesh,
        scratch_types=[
            pltpu.VMEM((_SPW, _CPS, _CHUNK), jnp.int32),
            pltpu.VMEM((_STRIP_B, _EMB_DIM), jnp.float32),
            pltpu.VMEM((_STRIP_B, _EMB_DIM), jnp.float32),
            pltpu.VMEM((_BPW, _N_NUM), jnp.float32),
            pltpu.SemaphoreType.DMA,
            pltpu.SemaphoreType.DMA,
        ],
        compiler_params=pltpu.CompilerParams(use_tc_tiling_on_sc=False),
    )
    def k(tbl_hbm, idx_hbm, cont_hbm, out_hbm,
          idx_v, buf0, buf1, cvs, sem0, sem1):
        wid = lax.axis_index("c") * _NS + lax.axis_index("s")
        pltpu.sync_copy(idx_hbm.at[pl.ds(wid * _SPW, _SPW)], idx_v)

        bufs = (buf0, buf1)
        sems = (sem0, sem1)

        def fire(sl, p):
            for j in range(_CPS):
                pltpu.async_copy(
                    tbl_hbm.at[idx_v.at[sl, j]],
                    bufs[p].at[pl.ds(j * _CHUNK, _CHUNK)],
                    sems[p])

        def drain(sl, p):
            for j in range(_CPS):
                pltpu.make_async_copy(
                    tbl_hbm.at[idx_v.at[sl, j]],
                    bufs[p].at[pl.ds(j * _CHUNK, _CHUNK)],
                    sems[p]).wait()

        def wout(sl, p):
            s = wid * _SPW + sl
            f = s // _SPF
            b0 = (s - f * _SPF) * _STRIP_B
            pltpu.sync_copy(
                bufs[p],
                out_hbm.at[pl.ds(b0, _STRIP_B), pl.ds(f * _EMB_DIM, _EMB_DIM)])

        fire(0, 0)

        base = wid * _BPW
        pltpu.sync_copy(cont_hbm.at[pl.ds(base, _BPW)], cvs)
        pltpu.sync_copy(cvs, out_hbm.at[pl.ds(base, _BPW), pl.ds(_EMB_D, _N_NUM)])

        def strip_pair(h, carry):
            s0 = 2 * h
            fire(s0 + 1, 1)
            drain(s0, 0)
            wout(s0, 0)
            fire(s0 + 2, 0)
            drain(s0 + 1, 1)
            wout(s0 + 1, 1)
            return carry

        lax.fori_loop(0, (_SPW - 1) // 2, strip_pair, 0)
        drain(_SPW - 1, 0)
        wout(_SPW - 1, 0)

    return k(tables_flat, idx, cont)


def kernel(x_numerical, x_cat, tables, gamma, beta):
    idx = (x_cat.T.astype(jnp.int32)
           + jnp.arange(_N_FIELDS, dtype=jnp.int32)[:, None] * _VOCAB)
    idx = idx.reshape(_N_STRIPS, _CPS, _CHUNK)
    tables_flat = tables.reshape(_N_FIELDS * _VOCAB, _EMB_DIM)

    scale, shift = _bn_stats(x_numerical, gamma, beta)
    cont = _bn_apply(x_numerical, scale, shift)

    return _sc_fused(tables_flat, idx, cont)
